# baseline (speedup 1.0000x reference)
import functools

import jax
import jax.numpy as jnp
from jax import lax
from jax.experimental import pallas as pl
from jax.experimental.pallas import tpu as pltpu
from jax.experimental.pallas import tpu_sc as plsc

N = 100000
E = 1600000
VOCAB = 65536
EMB = 32
HID = 32
G = 1024

NP = 102400
EP = 1605632
GP = 2048
ER = EP // 128
NR = NP // 128
TN = NP // 16
TG = GP // 16

_MESH = functools.partial(
    plsc.VectorSubcoreMesh, core_axis_name="c", subcore_axis_name="s")

_SC_PARAMS = pltpu.CompilerParams(use_tc_tiling_on_sc=False,
                                  skip_device_barrier=True)

_f32 = jnp.float32
_i32 = jnp.int32


def _m8(v):
    return pl.multiple_of(v, 8)


def _sc_stage_a(x2d, dst2d, batch2d, emb, znp, ones_h):
    def body(x_h, dst_h, b_h, emb_h, znp_h, ones_hbm,
             deg_out, cnt_out, h_out,
             idx, idxx, idxb, ones_v, hb, gsem, wsem, ssem,
             deg_sh, cnt_sh):
        c = lax.axis_index("c")
        s = lax.axis_index("s")
        w = c * 16 + s
        pltpu.sync_copy(znp_h.at[pl.ds(_m8(s * TN), TN)],
                        deg_sh.at[pl.ds(_m8(s * TN), TN)])
        pltpu.sync_copy(znp_h.at[pl.ds(_m8(s * TG), TG)],
                        cnt_sh.at[pl.ds(_m8(s * TG), TG)])
        pltpu.sync_copy(ones_hbm, ones_v)
        plsc.subcore_barrier()

        def deg_blk(i, _):
            r0 = _m8(c * (ER // 2) + s * (ER // 32) + i * 8)
            pltpu.sync_copy(dst_h.at[pl.ds(r0, 8)], idx)
            ds_ = [pltpu.async_copy(ones_v, deg_sh.at[idx.at[j]], ssem, add=True)
                   for j in range(8)]
            for d in ds_:
                d.wait()
            return 0
        lax.fori_loop(0, (ER // 32) // 8, deg_blk, 0)

        for k in range((NR // 8 + 31) // 32):
            b = w + k * 32
            @pl.when(b < NR // 8)
            def _():
                pltpu.sync_copy(x_h.at[pl.ds(_m8(b * 8), 8)], idxx)
                pltpu.sync_copy(b_h.at[pl.ds(_m8(b * 8), 8)], idxb)
                gd = [pltpu.async_copy(emb_h.at[idxx.at[j]], hb.at[j], gsem)
                      for j in range(8)]
                for d in gd:
                    d.wait()
                wd = [pltpu.async_copy(
                          hb.at[j],
                          h_out.at[pl.ds(_m8((b * 8 + j) * 128), 128)],
                          wsem)
                      for j in range(8)]
                for d in wd:
                    d.wait()
                for j in range(8):
                    pltpu.sync_copy(ones_v, cnt_sh.at[idxb.at[j]], add=True)

        plsc.subcore_barrier()
        pltpu.sync_copy(deg_sh.at[pl.ds(_m8(s * TN), TN)],
                        deg_out.at[pl.ds(_m8(c * NP + s * TN), TN)])
        pltpu.sync_copy(cnt_sh.at[pl.ds(_m8(s * TG), TG)],
                        cnt_out.at[pl.ds(_m8(c * GP + s * TG), TG)])

    return pl.kernel(
        body,
        out_type=(
            jax.ShapeDtypeStruct((2 * NP,), _f32),
            jax.ShapeDtypeStruct((2 * GP,), _f32),
            jax.ShapeDtypeStruct((NP, EMB), _f32),
        ),
        mesh=_MESH(),
        compiler_params=_SC_PARAMS,
        scratch_types=[
            pltpu.VMEM((8, 128), _i32),
            pltpu.VMEM((8, 128), _i32),
            pltpu.VMEM((8, 128), _i32),
            pltpu.VMEM((128,), _f32),
            pltpu.VMEM((8, 128, EMB), _f32),
            pltpu.SemaphoreType.DMA,
            pltpu.SemaphoreType.DMA,
            pltpu.SemaphoreType.DMA,
            pltpu.VMEM_SHARED((NP,), _f32),
            pltpu.VMEM_SHARED((GP,), _f32),
        ],
    )(x2d, dst2d, batch2d, emb, znp, ones_h)


_EC = 512
_ERC = EP // _EC
_TC_ROWS = _ERC // 16


def _sc_edge_pass(u2d, srcoff, dst2d):
    def body(u_h, src_h, dst_h, a_out, idxs, idxd, rows, gsem, ssem, s_sh):
        c = lax.axis_index("c")
        s = lax.axis_index("s")
        base = _m8(s * TN)
        pltpu.sync_copy(u_h.at[pl.ds(_m8(c * NP + base), TN)],
                        s_sh.at[pl.ds(base, TN)])
        plsc.subcore_barrier()

        def blk(i, _):
            r0 = s * _TC_ROWS + i * 2
            pltpu.sync_copy(src_h.at[pl.ds(c * _ERC + r0, 2)], idxs)
            pltpu.sync_copy(dst_h.at[pl.ds(r0, 2)], idxd)
            g0 = pltpu.async_copy(u_h.at[idxs.at[0]], rows.at[0], gsem)
            g1 = pltpu.async_copy(u_h.at[idxs.at[1]], rows.at[1], gsem)
            g0.wait()
            s0 = pltpu.async_copy(rows.at[0], s_sh.at[idxd.at[0]], ssem,
                                  add=True)
            g1.wait()
            s1 = pltpu.async_copy(rows.at[1], s_sh.at[idxd.at[1]], ssem,
                                  add=True)
            s0.wait()
            s1.wait()
            return 0
        lax.fori_loop(0, _TC_ROWS // 2, blk, 0)

        plsc.subcore_barrier()
        pltpu.sync_copy(s_sh.at[pl.ds(base, TN)],
                        a_out.at[pl.ds(_m8(c * NP + base), TN)])

    return pl.kernel(
        body,
        out_type=jax.ShapeDtypeStruct((2 * NP, 16), _f32),
        mesh=_MESH(),
        compiler_params=_SC_PARAMS,
        scratch_types=[
            pltpu.VMEM((2, _EC), _i32),
            pltpu.VMEM((2, _EC), _i32),
            pltpu.VMEM((2, _EC, 16), _f32),
            pltpu.SemaphoreType.DMA,
            pltpu.SemaphoreType.DMA,
            pltpu.VMEM_SHARED((NP, 16), _f32),
        ],
    )(u2d, srcoff, dst2d)


def _sc_pool(v2d, batch2d, zgp):
    def body(v_h, b_h, z_h, pool_out, idxb, vb, gsem, ssem, p_sh):
        c = lax.axis_index("c")
        s = lax.axis_index("s")
        pltpu.sync_copy(z_h.at[pl.ds(_m8(s * TG), TG)],
                        p_sh.at[pl.ds(_m8(s * TG), TG)])
        plsc.subcore_barrier()

        for k in range((NR // 8 + 15) // 16):
            b = s + k * 16
            @pl.when(b < NR // 8)
            def _():
                pltpu.sync_copy(b_h.at[pl.ds(_m8(b * 8), 8)], idxb)
                gd = [pltpu.async_copy(
                          v_h.at[pl.ds(_m8(c * NP + (b * 8 + j) * 128), 128)],
                          vb.at[j], gsem)
                      for j in range(8)]
                for d in gd:
                    d.wait()
                sd = [pltpu.async_copy(vb.at[j], p_sh.at[idxb.at[j]], ssem,
                                       add=True)
                      for j in range(8)]
                for d in sd:
                    d.wait()

        plsc.subcore_barrier()
        pltpu.sync_copy(p_sh.at[pl.ds(_m8(s * TG), TG)],
                        pool_out.at[pl.ds(_m8(c * GP + s * TG), TG)])

    return pl.kernel(
        body,
        out_type=jax.ShapeDtypeStruct((2 * GP, 16), _f32),
        mesh=_MESH(),
        compiler_params=_SC_PARAMS,
        scratch_types=[
            pltpu.VMEM((8, 128), _i32),
            pltpu.VMEM((8, 128, 16), _f32),
            pltpu.SemaphoreType.DMA,
            pltpu.SemaphoreType.DMA,
            pltpu.VMEM_SHARED((GP, 16), _f32),
        ],
    )(v2d, batch2d, zgp)


_BLK = 2048


def _tc_b(deg2, h):
    def body(d_ref, h_ref, u_ref):
        dinv = lax.rsqrt(d_ref[0, :] + d_ref[1, :] + 1.0)
        u = h_ref[...] * dinv[:, None]
        u_ref[0] = u[:, :16]
        u_ref[1] = u[:, 16:]

    return pl.pallas_call(
        body,
        grid=(NP // _BLK,),
        in_specs=[
            pl.BlockSpec((2, _BLK), lambda i: (0, i)),
            pl.BlockSpec((_BLK, EMB), lambda i: (i, 0)),
        ],
        out_specs=pl.BlockSpec((2, _BLK, 16), lambda i: (0, i, 0)),
        out_shape=jax.ShapeDtypeStruct((2, NP, 16), _f32),
    )(deg2, h)


def _tc_d(deg2, a1, W1, b1):
    def body(d_ref, a_ref, w_ref, b_ref, u_ref):
        dinv = lax.rsqrt(d_ref[0, :] + d_ref[1, :] + 1.0)
        af = jnp.concatenate([a_ref[0], a_ref[1]], axis=1)
        z = jnp.dot(af, w_ref[...], preferred_element_type=_f32)
        z = jnp.maximum(z * dinv[:, None] + b_ref[...], 0.0)
        u2 = z * dinv[:, None]
        u_ref[0] = u2[:, :16]
        u_ref[1] = u2[:, 16:]

    return pl.pallas_call(
        body,
        grid=(NP // _BLK,),
        in_specs=[
            pl.BlockSpec((2, _BLK), lambda i: (0, i)),
            pl.BlockSpec((2, _BLK, 16), lambda i: (0, i, 0)),
            pl.BlockSpec((EMB, HID), lambda i: (0, 0)),
            pl.BlockSpec((1, HID), lambda i: (0, 0)),
        ],
        out_specs=pl.BlockSpec((2, _BLK, 16), lambda i: (0, i, 0)),
        out_shape=jax.ShapeDtypeStruct((2, NP, 16), _f32),
    )(deg2, a1, W1, b1)


def _tc_f(deg2, a2):
    def body(d_ref, a_ref, v_ref):
        dinv = lax.rsqrt(d_ref[0, :] + d_ref[1, :] + 1.0)
        v_ref[0] = a_ref[0] * dinv[:, None]
        v_ref[1] = a_ref[1] * dinv[:, None]

    return pl.pallas_call(
        body,
        grid=(NP // _BLK,),
        in_specs=[
            pl.BlockSpec((2, _BLK), lambda i: (0, i)),
            pl.BlockSpec((2, _BLK, 16), lambda i: (0, i, 0)),
        ],
        out_specs=pl.BlockSpec((2, _BLK, 16), lambda i: (0, i, 0)),
        out_shape=jax.ShapeDtypeStruct((2, NP, 16), _f32),
    )(deg2, a2)


def _tc_h(pool2, cnt2, W2, b2):
    def body(p_ref, c_ref, w_ref, b_ref, o_ref):
        pf = jnp.concatenate([p_ref[0, :G, :], p_ref[1, :G, :]], axis=1)
        cnt = c_ref[0, :G] + c_ref[1, :G]
        r = jnp.dot(pf, w_ref[...], preferred_element_type=_f32)
        r = r / jnp.maximum(cnt, 1.0)[:, None] + b_ref[...]
        o_ref[...] = jnp.where(cnt[:, None] > 0, r, 0.0)

    return pl.pallas_call(
        body,
        in_specs=[
            pl.BlockSpec((2, GP, 16), lambda: (0, 0, 0)),
            pl.BlockSpec((2, GP), lambda: (0, 0)),
            pl.BlockSpec((HID, HID), lambda: (0, 0)),
            pl.BlockSpec((1, HID), lambda: (0, 0)),
        ],
        out_specs=pl.BlockSpec((G, HID), lambda: (0, 0)),
        out_shape=jax.ShapeDtypeStruct((G, HID), _f32),
    )(pool2, cnt2, W2, b2)


def kernel(x, edge_index, batch, emb, W1, b1, W2, b2):
    x = x.astype(_i32)
    src = edge_index[0].astype(_i32)
    dst = edge_index[1].astype(_i32)
    batch = batch.astype(_i32)

    epad = jnp.full((EP - E,), N, _i32)
    src2d = jnp.concatenate([src, epad]).reshape(ER, 128)
    dst2d = jnp.concatenate([dst, epad]).reshape(ER, 128)
    srcoff = jnp.concatenate([src2d, src2d + NP], axis=0)
    x2d = jnp.concatenate([x, jnp.zeros((NP - N,), _i32)]).reshape(NR, 128)
    b2d = jnp.concatenate([batch, jnp.full((NP - N,), G, _i32)]).reshape(NR, 128)

    znp = jnp.zeros((NP,), _f32)
    zgp = jnp.zeros((GP, 16), _f32)
    ones_h = jnp.ones((128,), _f32)
    b1r = b1.reshape(1, HID)
    b2r = b2.reshape(1, HID)

    deg2, cnt2, h = _sc_stage_a(x2d, dst2d, b2d, emb, znp, ones_h)
    deg2 = deg2.reshape(2, NP)

    srcoff_w = srcoff.reshape(2 * _ERC, _EC)
    dst_w = dst2d.reshape(_ERC, _EC)
    u1 = _tc_b(deg2, h).reshape(2 * NP, 16)
    a1 = _sc_edge_pass(u1, srcoff_w, dst_w).reshape(2, NP, 16)
    u2 = _tc_d(deg2, a1, W1, b1r).reshape(2 * NP, 16)
    a2 = _sc_edge_pass(u2, srcoff_w, dst_w).reshape(2, NP, 16)
    v = _tc_f(deg2, a2).reshape(2 * NP, 16)
    pool2 = _sc_pool(v, b2d, zgp).reshape(2, GP, 16)
    out = _tc_h(pool2, cnt2.reshape(2, GP), W2, b2r)
    return out


# speedup vs baseline: 1.0006x; 1.0006x over previous
import functools

import jax
import jax.numpy as jnp
from jax import lax
from jax.experimental import pallas as pl
from jax.experimental.pallas import tpu as pltpu
from jax.experimental.pallas import tpu_sc as plsc

N = 100000
E = 1600000
VOCAB = 65536
EMB = 32
HID = 32
G = 1024

NP = 102400
EP = 1605632
GP = 2048
ER = EP // 128
NR = NP // 128
TN = NP // 16
TG = GP // 16

_MESH = functools.partial(
    plsc.VectorSubcoreMesh, core_axis_name="c", subcore_axis_name="s")

_SC_PARAMS = pltpu.CompilerParams(use_tc_tiling_on_sc=False)

_f32 = jnp.float32
_i32 = jnp.int32


def _m8(v):
    return pl.multiple_of(v, 8)


def _sc_stage_a(x2d, dst2d, batch2d, emb, znp, ones_h):
    def body(x_h, dst_h, b_h, emb_h, znp_h, ones_hbm,
             deg_out, cnt_out, h_out,
             idx, idxx, idxb, ones_v, hb, gsem, wsem, ssem,
             deg_sh, cnt_sh):
        c = lax.axis_index("c")
        s = lax.axis_index("s")
        w = c * 16 + s
        pltpu.sync_copy(znp_h.at[pl.ds(_m8(s * TN), TN)],
                        deg_sh.at[pl.ds(_m8(s * TN), TN)])
        pltpu.sync_copy(znp_h.at[pl.ds(_m8(s * TG), TG)],
                        cnt_sh.at[pl.ds(_m8(s * TG), TG)])
        pltpu.sync_copy(ones_hbm, ones_v)
        plsc.subcore_barrier()

        def deg_blk(i, _):
            r0 = _m8(c * (ER // 2) + s * (ER // 32) + i * 8)
            pltpu.sync_copy(dst_h.at[pl.ds(r0, 8)], idx)
            ds_ = [pltpu.async_copy(ones_v, deg_sh.at[idx.at[j]], ssem, add=True)
                   for j in range(8)]
            for d in ds_:
                d.wait()
            return 0
        lax.fori_loop(0, (ER // 32) // 8, deg_blk, 0)

        for k in range((NR // 8 + 31) // 32):
            b = w + k * 32
            @pl.when(b < NR // 8)
            def _():
                pltpu.sync_copy(x_h.at[pl.ds(_m8(b * 8), 8)], idxx)
                pltpu.sync_copy(b_h.at[pl.ds(_m8(b * 8), 8)], idxb)
                gd = [pltpu.async_copy(emb_h.at[idxx.at[j]], hb.at[j], gsem)
                      for j in range(8)]
                for d in gd:
                    d.wait()
                wd = [pltpu.async_copy(
                          hb.at[j],
                          h_out.at[pl.ds(_m8((b * 8 + j) * 128), 128)],
                          wsem)
                      for j in range(8)]
                for d in wd:
                    d.wait()
                for j in range(8):
                    pltpu.sync_copy(ones_v, cnt_sh.at[idxb.at[j]], add=True)

        plsc.subcore_barrier()
        pltpu.sync_copy(deg_sh.at[pl.ds(_m8(s * TN), TN)],
                        deg_out.at[pl.ds(_m8(c * NP + s * TN), TN)])
        pltpu.sync_copy(cnt_sh.at[pl.ds(_m8(s * TG), TG)],
                        cnt_out.at[pl.ds(_m8(c * GP + s * TG), TG)])

    return pl.kernel(
        body,
        out_type=(
            jax.ShapeDtypeStruct((2 * NP,), _f32),
            jax.ShapeDtypeStruct((2 * GP,), _f32),
            jax.ShapeDtypeStruct((NP, EMB), _f32),
        ),
        mesh=_MESH(),
        compiler_params=_SC_PARAMS,
        scratch_types=[
            pltpu.VMEM((8, 128), _i32),
            pltpu.VMEM((8, 128), _i32),
            pltpu.VMEM((8, 128), _i32),
            pltpu.VMEM((128,), _f32),
            pltpu.VMEM((8, 128, EMB), _f32),
            pltpu.SemaphoreType.DMA,
            pltpu.SemaphoreType.DMA,
            pltpu.SemaphoreType.DMA,
            pltpu.VMEM_SHARED((NP,), _f32),
            pltpu.VMEM_SHARED((GP,), _f32),
        ],
    )(x2d, dst2d, batch2d, emb, znp, ones_h)


_EC = 512
_ERC = EP // _EC
_TC_ROWS = _ERC // 16


def _sc_edge_pass(u2d, srcoff, dst2d):
    def body(u_h, src_h, dst_h, a_out, idxs, idxd, rows, gsem, ssem, s_sh):
        c = lax.axis_index("c")
        s = lax.axis_index("s")
        base = _m8(s * TN)
        pltpu.sync_copy(u_h.at[pl.ds(_m8(c * NP + base), TN)],
                        s_sh.at[pl.ds(base, TN)])
        plsc.subcore_barrier()

        def blk(i, _):
            r0 = s * _TC_ROWS + i * 2
            pltpu.sync_copy(src_h.at[pl.ds(c * _ERC + r0, 2)], idxs)
            pltpu.sync_copy(dst_h.at[pl.ds(r0, 2)], idxd)
            g0 = pltpu.async_copy(u_h.at[idxs.at[0]], rows.at[0], gsem)
            g1 = pltpu.async_copy(u_h.at[idxs.at[1]], rows.at[1], gsem)
            g0.wait()
            s0 = pltpu.async_copy(rows.at[0], s_sh.at[idxd.at[0]], ssem,
                                  add=True)
            g1.wait()
            s1 = pltpu.async_copy(rows.at[1], s_sh.at[idxd.at[1]], ssem,
                                  add=True)
            s0.wait()
            s1.wait()
            return 0
        lax.fori_loop(0, _TC_ROWS // 2, blk, 0)

        plsc.subcore_barrier()
        pltpu.sync_copy(s_sh.at[pl.ds(base, TN)],
                        a_out.at[pl.ds(_m8(c * NP + base), TN)])

    return pl.kernel(
        body,
        out_type=jax.ShapeDtypeStruct((2 * NP, 16), _f32),
        mesh=_MESH(),
        compiler_params=_SC_PARAMS,
        scratch_types=[
            pltpu.VMEM((2, _EC), _i32),
            pltpu.VMEM((2, _EC), _i32),
            pltpu.VMEM((2, _EC, 16), _f32),
            pltpu.SemaphoreType.DMA,
            pltpu.SemaphoreType.DMA,
            pltpu.VMEM_SHARED((NP, 16), _f32),
        ],
    )(u2d, srcoff, dst2d)


def _sc_pool(v2d, batch2d, zgp):
    def body(v_h, b_h, z_h, pool_out, idxb, vb, gsem, ssem, p_sh):
        c = lax.axis_index("c")
        s = lax.axis_index("s")
        pltpu.sync_copy(z_h.at[pl.ds(_m8(s * TG), TG)],
                        p_sh.at[pl.ds(_m8(s * TG), TG)])
        plsc.subcore_barrier()

        for k in range((NR // 8 + 15) // 16):
            b = s + k * 16
            @pl.when(b < NR // 8)
            def _():
                pltpu.sync_copy(b_h.at[pl.ds(_m8(b * 8), 8)], idxb)
                gd = [pltpu.async_copy(
                          v_h.at[pl.ds(_m8(c * NP + (b * 8 + j) * 128), 128)],
                          vb.at[j], gsem)
                      for j in range(8)]
                for d in gd:
                    d.wait()
                sd = [pltpu.async_copy(vb.at[j], p_sh.at[idxb.at[j]], ssem,
                                       add=True)
                      for j in range(8)]
                for d in sd:
                    d.wait()

        plsc.subcore_barrier()
        pltpu.sync_copy(p_sh.at[pl.ds(_m8(s * TG), TG)],
                        pool_out.at[pl.ds(_m8(c * GP + s * TG), TG)])

    return pl.kernel(
        body,
        out_type=jax.ShapeDtypeStruct((2 * GP, 16), _f32),
        mesh=_MESH(),
        compiler_params=_SC_PARAMS,
        scratch_types=[
            pltpu.VMEM((8, 128), _i32),
            pltpu.VMEM((8, 128, 16), _f32),
            pltpu.SemaphoreType.DMA,
            pltpu.SemaphoreType.DMA,
            pltpu.VMEM_SHARED((GP, 16), _f32),
        ],
    )(v2d, batch2d, zgp)


_BLK = 2048


def _tc_b(deg2, h):
    def body(d_ref, h_ref, u_ref):
        dinv = lax.rsqrt(d_ref[0, :] + d_ref[1, :] + 1.0)
        u = h_ref[...] * dinv[:, None]
        u_ref[0] = u[:, :16]
        u_ref[1] = u[:, 16:]

    return pl.pallas_call(
        body,
        grid=(NP // _BLK,),
        in_specs=[
            pl.BlockSpec((2, _BLK), lambda i: (0, i)),
            pl.BlockSpec((_BLK, EMB), lambda i: (i, 0)),
        ],
        out_specs=pl.BlockSpec((2, _BLK, 16), lambda i: (0, i, 0)),
        out_shape=jax.ShapeDtypeStruct((2, NP, 16), _f32),
    )(deg2, h)


def _tc_d(deg2, a1, W1, b1):
    def body(d_ref, a_ref, w_ref, b_ref, u_ref):
        dinv = lax.rsqrt(d_ref[0, :] + d_ref[1, :] + 1.0)
        af = jnp.concatenate([a_ref[0], a_ref[1]], axis=1)
        z = jnp.dot(af, w_ref[...], preferred_element_type=_f32)
        z = jnp.maximum(z * dinv[:, None] + b_ref[...], 0.0)
        u2 = z * dinv[:, None]
        u_ref[0] = u2[:, :16]
        u_ref[1] = u2[:, 16:]

    return pl.pallas_call(
        body,
        grid=(NP // _BLK,),
        in_specs=[
            pl.BlockSpec((2, _BLK), lambda i: (0, i)),
            pl.BlockSpec((2, _BLK, 16), lambda i: (0, i, 0)),
            pl.BlockSpec((EMB, HID), lambda i: (0, 0)),
            pl.BlockSpec((1, HID), lambda i: (0, 0)),
        ],
        out_specs=pl.BlockSpec((2, _BLK, 16), lambda i: (0, i, 0)),
        out_shape=jax.ShapeDtypeStruct((2, NP, 16), _f32),
    )(deg2, a1, W1, b1)


def _tc_f(deg2, a2):
    def body(d_ref, a_ref, v_ref):
        dinv = lax.rsqrt(d_ref[0, :] + d_ref[1, :] + 1.0)
        v_ref[0] = a_ref[0] * dinv[:, None]
        v_ref[1] = a_ref[1] * dinv[:, None]

    return pl.pallas_call(
        body,
        grid=(NP // _BLK,),
        in_specs=[
            pl.BlockSpec((2, _BLK), lambda i: (0, i)),
            pl.BlockSpec((2, _BLK, 16), lambda i: (0, i, 0)),
        ],
        out_specs=pl.BlockSpec((2, _BLK, 16), lambda i: (0, i, 0)),
        out_shape=jax.ShapeDtypeStruct((2, NP, 16), _f32),
    )(deg2, a2)


def _tc_h(pool2, cnt2, W2, b2):
    def body(p_ref, c_ref, w_ref, b_ref, o_ref):
        pf = jnp.concatenate([p_ref[0, :G, :], p_ref[1, :G, :]], axis=1)
        cnt = c_ref[0, :G] + c_ref[1, :G]
        r = jnp.dot(pf, w_ref[...], preferred_element_type=_f32)
        r = r / jnp.maximum(cnt, 1.0)[:, None] + b_ref[...]
        o_ref[...] = jnp.where(cnt[:, None] > 0, r, 0.0)

    return pl.pallas_call(
        body,
        in_specs=[
            pl.BlockSpec((2, GP, 16), lambda: (0, 0, 0)),
            pl.BlockSpec((2, GP), lambda: (0, 0)),
            pl.BlockSpec((HID, HID), lambda: (0, 0)),
            pl.BlockSpec((1, HID), lambda: (0, 0)),
        ],
        out_specs=pl.BlockSpec((G, HID), lambda: (0, 0)),
        out_shape=jax.ShapeDtypeStruct((G, HID), _f32),
    )(pool2, cnt2, W2, b2)


def kernel(x, edge_index, batch, emb, W1, b1, W2, b2):
    x = x.astype(_i32)
    src = edge_index[0].astype(_i32)
    dst = edge_index[1].astype(_i32)
    batch = batch.astype(_i32)

    epad = jnp.full((EP - E,), N, _i32)
    src2d = jnp.concatenate([src, epad]).reshape(ER, 128)
    dst2d = jnp.concatenate([dst, epad]).reshape(ER, 128)
    srcoff = jnp.concatenate([src2d, src2d + NP], axis=0)
    x2d = jnp.concatenate([x, jnp.zeros((NP - N,), _i32)]).reshape(NR, 128)
    b2d = jnp.concatenate([batch, jnp.full((NP - N,), G, _i32)]).reshape(NR, 128)

    znp = jnp.zeros((NP,), _f32)
    zgp = jnp.zeros((GP, 16), _f32)
    ones_h = jnp.ones((128,), _f32)
    b1r = b1.reshape(1, HID)
    b2r = b2.reshape(1, HID)

    deg2, cnt2, h = _sc_stage_a(x2d, dst2d, b2d, emb, znp, ones_h)
    deg2 = deg2.reshape(2, NP)

    srcoff_w = srcoff.reshape(2 * _ERC, _EC)
    dst_w = dst2d.reshape(_ERC, _EC)
    u1 = _tc_b(deg2, h).reshape(2 * NP, 16)
    a1 = _sc_edge_pass(u1, srcoff_w, dst_w).reshape(2, NP, 16)
    u2 = _tc_d(deg2, a1, W1, b1r).reshape(2 * NP, 16)
    a2 = _sc_edge_pass(u2, srcoff_w, dst_w).reshape(2, NP, 16)
    v = _tc_f(deg2, a2).reshape(2 * NP, 16)
    pool2 = _sc_pool(v, b2d, zgp).reshape(2, GP, 16)
    out = _tc_h(pool2, cnt2.reshape(2, GP), W2, b2r)
    return out
